# baseline (speedup 1.0000x reference)
import functools

import jax
import jax.numpy as jnp
from jax import lax
from jax.experimental import pallas as pl
from jax.experimental.pallas import tpu as pltpu
from jax.experimental.pallas import tpu_sc as plsc

_VOCAB = 1000000
_EMBED = 32
_BATCH = 16384
_C = 5

_NC = 2
_NS = 16
_NW = _NC * _NS
_BW = _BATCH // _NW
_CW = _BW * _C
_CHUNK = 128
_GB = 16


def _body(tgt_hbm, ctx_hbm, ttab_hbm, ctab_hbm, out_hbm,
          idx_t, idx_c, rows_t, rows_c, out_v, sem):
  wid = lax.axis_index("s") * _NC + lax.axis_index("c")

  pltpu.sync_copy(tgt_hbm.at[wid], idx_t)
  pltpu.sync_copy(ctx_hbm.at[wid], idx_c)

  copies = []
  for j in range(_BW // _CHUNK):
    copies.append(pltpu.async_copy(
        ttab_hbm.at[idx_t.at[j]],
        rows_t.at[pl.ds(j * _CHUNK, _CHUNK)], sem))
  for j in range(_CW // _CHUNK):
    copies.append(pltpu.async_copy(
        ctab_hbm.at[idx_c.at[j]],
        rows_c.at[pl.ds(j * _CHUNK, _CHUNK)], sem))
  for cp in copies:
    cp.wait()

  lanes = lax.iota(jnp.int32, 16)
  masks = [lanes == l for l in range(16)]

  def step(i, _):
    accs = [jnp.zeros((16,), jnp.float32) for _ in range(_C)]
    for k in range(_GB):
      b = i * _GB + k
      w0 = rows_t[b, pl.ds(0, 16)]
      w1 = rows_t[b, pl.ds(16, 16)]
      for c in range(_C):
        r = b * _C + c
        p = w0 * rows_c[r, pl.ds(0, 16)] + w1 * rows_c[r, pl.ds(16, 16)]
        s = jnp.sum(p)
        q = k * _C + c
        accs[q // 16] = jnp.where(masks[q % 16], s, accs[q // 16])
    base = i * (_GB * _C)
    for j in range(_C):
      out_v[pl.ds(base + j * 16, 16)] = accs[j]
    return ()

  lax.fori_loop(0, _BW // _GB, step, ())

  pltpu.sync_copy(out_v, out_hbm.at[wid])


@jax.jit
def _run(tgt3d, ctx3d, target_table, context_table):
  mesh = plsc.VectorSubcoreMesh(core_axis_name="c", subcore_axis_name="s")
  k = functools.partial(
      pl.kernel,
      mesh=mesh,
      compiler_params=pltpu.CompilerParams(
          use_tc_tiling_on_sc=False, needs_layout_passes=False),
      out_type=jax.ShapeDtypeStruct((_NW, _CW), jnp.float32),
      scratch_types=[
          pltpu.VMEM((_BW // _CHUNK, _CHUNK), jnp.int32),
          pltpu.VMEM((_CW // _CHUNK, _CHUNK), jnp.int32),
          pltpu.VMEM((_BW, _EMBED), jnp.float32),
          pltpu.VMEM((_CW, _EMBED), jnp.float32),
          pltpu.VMEM((_CW,), jnp.float32),
          pltpu.SemaphoreType.DMA,
      ],
  )(_body)
  return k(tgt3d, ctx3d, target_table, context_table)


def kernel(target, context, target_table, context_table):
  tgt3d = target.reshape(_NW, _BW // _CHUNK, _CHUNK)
  ctx3d = context.reshape(_NW, _CW // _CHUNK, _CHUNK)
  out = _run(tgt3d, ctx3d, target_table, context_table)
  return out.reshape(_BATCH, _C)
